# Initial kernel scaffold; baseline (speedup 1.0000x reference)
import jax
import jax.numpy as jnp
from jax.experimental import pallas as pl


def kernel(uv_, params):
    raise NotImplementedError("write your pallas kernel here")


# speedup vs baseline: 1.1896x; 1.1896x over previous
import functools

import jax
import jax.numpy as jnp
from jax import lax
from jax.experimental import pallas as pl
from jax.experimental.pallas import tpu as pltpu
from jax.experimental.pallas import tpu_sc as plsc

_W = 1024
_H = 1024
_F = 16
_B = 262144
_NC = 2
_NS = 16
_NW = _NC * _NS
_PPW = _B // _NW
_CHUNK = 128
_NCHUNK = _PPW // _CHUNK
_G = _CHUNK // 16


@functools.partial(
    pl.kernel,
    out_type=jax.ShapeDtypeStruct((_B, _F), jnp.float32),
    mesh=plsc.VectorSubcoreMesh(core_axis_name="c", subcore_axis_name="s"),
    compiler_params=pltpu.CompilerParams(use_tc_tiling_on_sc=False),
    scratch_types=[
        pltpu.VMEM((_PPW,), jnp.float32),
        pltpu.VMEM((_PPW,), jnp.float32),
        pltpu.VMEM((_CHUNK,), jnp.int32),
        pltpu.VMEM((_CHUNK,), jnp.int32),
        pltpu.VMEM((_CHUNK,), jnp.int32),
        pltpu.VMEM((_CHUNK,), jnp.int32),
        pltpu.VMEM((_CHUNK,), jnp.float32),
        pltpu.VMEM((_CHUNK,), jnp.float32),
        pltpu.VMEM((_CHUNK, _F), jnp.float32),
        pltpu.VMEM((_CHUNK, _F), jnp.float32),
        pltpu.VMEM((_CHUNK, _F), jnp.float32),
        pltpu.VMEM((_CHUNK, _F), jnp.float32),
        pltpu.VMEM((_CHUNK, _F), jnp.float32),
        pltpu.SemaphoreType.DMA,
        pltpu.SemaphoreType.DMA,
        pltpu.SemaphoreType.DMA,
        pltpu.SemaphoreType.DMA,
    ],
)
def _sample(xs_hbm, ys_hbm, tab_hbm, out_hbm,
            xs_v, ys_v, i00_v, i01_v, i10_v, i11_v, fx_v, fy_v,
            g00, g01, g10, g11, out_v, sem0, sem1, sem2, sem3):
  wid = lax.axis_index("s") * _NC + lax.axis_index("c")
  base = wid * _PPW
  pltpu.sync_copy(xs_hbm.at[pl.ds(base, _PPW)], xs_v)
  pltpu.sync_copy(ys_hbm.at[pl.ds(base, _PPW)], ys_v)

  def chunk_body(c, carry):
    off = c * _CHUNK

    def idx_body(g, carry2):
      o = off + g * 16
      u = xs_v[pl.ds(o, 16)]
      v = ys_v[pl.ds(o, 16)]
      x = ((u * 2.0 - 1.0) + 1.0) * 0.5 * float(_W - 1)
      y = ((v * 2.0 - 1.0) + 1.0) * 0.5 * float(_H - 1)
      xi = x.astype(jnp.int32)
      yi = y.astype(jnp.int32)
      s = g * 16
      fx_v[pl.ds(s, 16)] = x - xi.astype(jnp.float32)
      fy_v[pl.ds(s, 16)] = y - yi.astype(jnp.float32)
      r00 = (yi << 10) + xi
      i00_v[pl.ds(s, 16)] = r00
      i01_v[pl.ds(s, 16)] = r00 + 1
      i10_v[pl.ds(s, 16)] = r00 + _W
      i11_v[pl.ds(s, 16)] = r00 + (_W + 1)
      return carry2

    lax.fori_loop(0, _G, idx_body, 0)

    cp0 = pltpu.async_copy(tab_hbm.at[i00_v], g00, sem0)
    cp1 = pltpu.async_copy(tab_hbm.at[i01_v], g01, sem1)
    cp2 = pltpu.async_copy(tab_hbm.at[i10_v], g10, sem2)
    cp3 = pltpu.async_copy(tab_hbm.at[i11_v], g11, sem3)
    cp0.wait()
    cp1.wait()
    cp2.wait()
    cp3.wait()

    def blend_body(g, carry2):
      s = g * 16
      fxg = fx_v[pl.ds(s, 16)]
      fyg = fy_v[pl.ds(s, 16)]
      for j in range(16):
        i = s + j
        a00 = g00[i, :]
        a01 = g01[i, :]
        a10 = g10[i, :]
        a11 = g11[i, :]
        fx = jnp.full((16,), fxg[j], jnp.float32)
        fy = jnp.full((16,), fyg[j], jnp.float32)
        top = a00 + fx * (a01 - a00)
        bot = a10 + fx * (a11 - a10)
        out_v[i, :] = top + fy * (bot - top)
      return carry2

    lax.fori_loop(0, _G, blend_body, 0)
    pltpu.sync_copy(out_v, out_hbm.at[pl.ds(base + off, _CHUNK)])
    return carry

  lax.fori_loop(0, _NCHUNK, chunk_body, 0)


def kernel(uv_, params):
  table = jnp.transpose(params[0], (1, 2, 0)).reshape(_H * _W, _F)
  xs = uv_[:, 0]
  ys = uv_[:, 1]
  return _sample(xs, ys, table)
